# baseline (speedup 1.0000x reference)
import jax
import jax.numpy as jnp
from jax.experimental import pallas as pl

BATCH = 4096
DIM = 3328
BLOCK_ROWS = 32


def _sum8_kernel(t0, t1, t2, t3, t4, t5, t6, t7, o):
    o[...] = (((t0[...] + t1[...]) + (t2[...] + t3[...]))
              + ((t4[...] + t5[...]) + (t6[...] + t7[...])))


def kernel(tensors_0, tensors_1, tensors_2, tensors_3, tensors_4, tensors_5, tensors_6, tensors_7):
    spec = pl.BlockSpec((BLOCK_ROWS, DIM), lambda i: (i, 0))
    return pl.pallas_call(
        _sum8_kernel,
        grid=(BATCH // BLOCK_ROWS,),
        in_specs=[spec] * 8,
        out_specs=spec,
        out_shape=jax.ShapeDtypeStruct((BATCH, DIM), jnp.float32),
    )(tensors_0, tensors_1, tensors_2, tensors_3,
      tensors_4, tensors_5, tensors_6, tensors_7)


# speedup vs baseline: 1.1099x; 1.1099x over previous
import jax
import jax.numpy as jnp
from jax.experimental import pallas as pl

BATCH = 4096
DIM = 3328
BLOCK_ROWS = 64


def _sum8_kernel(t0, t1, t2, t3, t4, t5, t6, t7, o):
    o[...] = (((t0[...] + t1[...]) + (t2[...] + t3[...]))
              + ((t4[...] + t5[...]) + (t6[...] + t7[...])))


def kernel(tensors_0, tensors_1, tensors_2, tensors_3, tensors_4, tensors_5, tensors_6, tensors_7):
    spec = pl.BlockSpec((BLOCK_ROWS, DIM), lambda i: (i, 0))
    return pl.pallas_call(
        _sum8_kernel,
        grid=(BATCH // BLOCK_ROWS,),
        in_specs=[spec] * 8,
        out_specs=spec,
        out_shape=jax.ShapeDtypeStruct((BATCH, DIM), jnp.float32),
    )(tensors_0, tensors_1, tensors_2, tensors_3,
      tensors_4, tensors_5, tensors_6, tensors_7)
